# initial kernel scaffold (unmeasured)
import jax
import jax.numpy as jnp
from jax import lax
from jax.experimental import pallas as pl
from jax.experimental.pallas import tpu as pltpu


def kernel(
    x,
):
    def body(*refs):
        pass

    out_shape = jax.ShapeDtypeStruct(..., jnp.float32)
    return pl.pallas_call(body, out_shape=out_shape)(...)


# baseline (device time: 419150 ns/iter reference)
import jax
import jax.numpy as jnp
from jax import lax
from jax.experimental import pallas as pl
from jax.experimental.pallas import tpu as pltpu

M, N = 8192, 1024
ADD_ROWS = 1024
N_ADD = M // ADD_ROWS


def kernel(x):
    def body(x_ref, out_ref, xbuf, copy_sem, send_sem, recv_sem):
        my_x = lax.axis_index("x")
        my_y = lax.axis_index("y")
        peer = (1 - my_x, my_y)

        barrier_sem = pltpu.get_barrier_semaphore()
        pl.semaphore_signal(
            barrier_sem, inc=1, device_id=peer,
            device_id_type=pl.DeviceIdType.MESH,
        )
        pl.semaphore_wait(barrier_sem, 1)

        rdma = pltpu.make_async_remote_copy(
            src_ref=x_ref,
            dst_ref=out_ref,
            send_sem=send_sem,
            recv_sem=recv_sem,
            device_id=peer,
            device_id_type=pl.DeviceIdType.MESH,
        )
        rdma.start()
        rdma.wait()

        for c in range(N_ADD):
            rows = pl.ds(c * ADD_ROWS, ADD_ROWS)
            cp = pltpu.make_async_copy(x_ref.at[rows, :], xbuf, copy_sem)
            cp.start()
            cp.wait()
            out_ref[rows, :] = out_ref[rows, :] + xbuf[:, :]

    return pl.pallas_call(
        body,
        out_shape=jax.ShapeDtypeStruct((M, N), jnp.float32),
        in_specs=[pl.BlockSpec(memory_space=pl.ANY)],
        out_specs=pl.BlockSpec(memory_space=pltpu.VMEM),
        scratch_shapes=[
            pltpu.VMEM((ADD_ROWS, N), jnp.float32),
            pltpu.SemaphoreType.DMA,
            pltpu.SemaphoreType.DMA,
            pltpu.SemaphoreType.DMA,
        ],
        compiler_params=pltpu.CompilerParams(
            collective_id=0,
            vmem_limit_bytes=56 * 1024 * 1024,
        ),
    )(x)


# device time: 233226 ns/iter; 1.7972x vs baseline; 1.7972x over previous
import jax
import jax.numpy as jnp
from jax import lax
from jax.experimental import pallas as pl
from jax.experimental.pallas import tpu as pltpu

M, N = 8192, 1024
HALF = M // 2
C = 16
R = HALF // C


def kernel(x):
    def body(x_ref, out_ref, stage, xbuf, copy_sem,
             sendx, recvx, sendf, recvf):
        my_x = lax.axis_index("x")
        my_y = lax.axis_index("y")
        xpeer = (1 - my_x, my_y)
        ypeer = (my_x, 1 - my_y)

        hd = my_y * HALF
        hf = (1 - my_y) * HALF

        barrier_sem = pltpu.get_barrier_semaphore()
        for nbr in (xpeer, ypeer):
            pl.semaphore_signal(
                barrier_sem, inc=1, device_id=nbr,
                device_id_type=pl.DeviceIdType.MESH,
            )
        pl.semaphore_wait(barrier_sem, 2)

        xsends = []
        for c in range(C):
            snd = pltpu.make_async_remote_copy(
                src_ref=x_ref.at[pl.ds(hd + c * R, R), :],
                dst_ref=stage.at[pl.ds(c * R, R), :],
                send_sem=sendx.at[c],
                recv_sem=recvx.at[c],
                device_id=xpeer,
                device_id_type=pl.DeviceIdType.MESH,
            )
            snd.start()
            xsends.append(snd)

        fins = [
            pltpu.make_async_remote_copy(
                src_ref=stage.at[pl.ds(c * R, R), :],
                dst_ref=out_ref.at[pl.ds(hf + c * R, R), :],
                send_sem=sendf.at[c],
                recv_sem=recvf.at[c],
                device_id=ypeer,
                device_id_type=pl.DeviceIdType.MESH,
            )
            for c in range(C)
        ]

        def add_forwarded(c):
            fins[c].wait_recv()
            cp = pltpu.make_async_copy(
                x_ref.at[pl.ds(hf + c * R, R), :], xbuf, copy_sem)
            cp.start()
            cp.wait()
            rows = pl.ds(hf + c * R, R)
            out_ref[rows, :] = out_ref[rows, :] + xbuf[:, :]

        fwds = []
        for c in range(C):
            xsends[c].wait_recv()
            fwd = pltpu.make_async_remote_copy(
                src_ref=stage.at[pl.ds(c * R, R), :],
                dst_ref=out_ref.at[pl.ds(hd + c * R, R), :],
                send_sem=sendf.at[c],
                recv_sem=recvf.at[c],
                device_id=ypeer,
                device_id_type=pl.DeviceIdType.MESH,
            )
            fwd.start()
            fwds.append(fwd)

            cp = pltpu.make_async_copy(
                x_ref.at[pl.ds(hd + c * R, R), :], xbuf, copy_sem)
            cp.start()
            cp.wait()
            rows = pl.ds(hd + c * R, R)
            out_ref[rows, :] = stage[pl.ds(c * R, R), :] + xbuf[:, :]

            if c >= 1:
                add_forwarded(c - 1)

        add_forwarded(C - 1)

        for c in range(C):
            xsends[c].wait_send()
            fwds[c].wait_send()

    return pl.pallas_call(
        body,
        out_shape=jax.ShapeDtypeStruct((M, N), jnp.float32),
        in_specs=[pl.BlockSpec(memory_space=pl.ANY)],
        out_specs=pl.BlockSpec(memory_space=pltpu.VMEM),
        scratch_shapes=[
            pltpu.VMEM((HALF, N), jnp.float32),
            pltpu.VMEM((R, N), jnp.float32),
            pltpu.SemaphoreType.DMA,
            pltpu.SemaphoreType.DMA((C,)),
            pltpu.SemaphoreType.DMA((C,)),
            pltpu.SemaphoreType.DMA((C,)),
            pltpu.SemaphoreType.DMA((C,)),
        ],
        compiler_params=pltpu.CompilerParams(
            collective_id=0,
            vmem_limit_bytes=56 * 1024 * 1024,
        ),
    )(x)


# device time: 216785 ns/iter; 1.9335x vs baseline; 1.0758x over previous
import jax
import jax.numpy as jnp
from jax import lax
from jax.experimental import pallas as pl
from jax.experimental.pallas import tpu as pltpu

M, N = 8192, 1024
HALF = M // 2

SIZES = [32, 96] + [128] * 30 + [96, 32]
assert sum(SIZES) == HALF
OFFS = [sum(SIZES[:i]) for i in range(len(SIZES))]
C = len(SIZES)
RMAX = max(SIZES)


def kernel(x):
    def body(x_ref, out_ref, stage, xbuf, copy_sems, store_sems,
             sendx, recvx, sendf, recvf):
        my_x = lax.axis_index("x")
        my_y = lax.axis_index("y")
        xpeer = (1 - my_x, my_y)
        ypeer = (my_x, 1 - my_y)

        hd = my_y * HALF
        hf = (1 - my_y) * HALF

        barrier_sem = pltpu.get_barrier_semaphore()
        for nbr in (xpeer, ypeer):
            pl.semaphore_signal(
                barrier_sem, inc=1, device_id=nbr,
                device_id_type=pl.DeviceIdType.MESH,
            )
        pl.semaphore_wait(barrier_sem, 2)

        xsends = []
        for c in range(C):
            snd = pltpu.make_async_remote_copy(
                src_ref=x_ref.at[pl.ds(hd + OFFS[c], SIZES[c]), :],
                dst_ref=stage.at[pl.ds(OFFS[c], SIZES[c]), :],
                send_sem=sendx.at[c],
                recv_sem=recvx.at[c],
                device_id=xpeer,
                device_id_type=pl.DeviceIdType.MESH,
            )
            snd.start()
            xsends.append(snd)

        fins = [
            pltpu.make_async_remote_copy(
                src_ref=stage.at[pl.ds(OFFS[c], SIZES[c]), :],
                dst_ref=out_ref.at[pl.ds(hf + OFFS[c], SIZES[c]), :],
                send_sem=sendf.at[c],
                recv_sem=recvf.at[c],
                device_id=ypeer,
                device_id_type=pl.DeviceIdType.MESH,
            )
            for c in range(C)
        ]

        def load_x(c):
            cp = pltpu.make_async_copy(
                x_ref.at[pl.ds(hd + OFFS[c], SIZES[c]), :],
                xbuf.at[c % 2, pl.ds(0, SIZES[c])], copy_sems.at[c % 2])
            cp.start()
            return cp

        loads = [load_x(0)]
        fwds = []
        stores = []
        for c in range(C):
            if c + 1 < C:
                loads.append(load_x(c + 1))
            xsends[c].wait_recv()
            loads[c].wait()
            crows = pl.ds(OFFS[c], SIZES[c])
            stage[crows, :] = stage[crows, :] + xbuf[c % 2, pl.ds(0, SIZES[c])]
            rows = pl.ds(hd + OFFS[c], SIZES[c])
            fwd = pltpu.make_async_remote_copy(
                src_ref=stage.at[crows, :],
                dst_ref=out_ref.at[rows, :],
                send_sem=sendf.at[c],
                recv_sem=recvf.at[c],
                device_id=ypeer,
                device_id_type=pl.DeviceIdType.MESH,
            )
            fwd.start()
            fwds.append(fwd)
            st = pltpu.make_async_copy(
                stage.at[crows, :], out_ref.at[rows, :], store_sems.at[c])
            st.start()
            stores.append(st)

        for c in range(C):
            fins[c].wait_recv()

        for c in range(C):
            stores[c].wait()
            xsends[c].wait_send()
            fwds[c].wait_send()

    return pl.pallas_call(
        body,
        out_shape=jax.ShapeDtypeStruct((M, N), jnp.float32),
        in_specs=[pl.BlockSpec(memory_space=pl.ANY)],
        out_specs=pl.BlockSpec(memory_space=pl.ANY),
        scratch_shapes=[
            pltpu.VMEM((HALF, N), jnp.float32),
            pltpu.VMEM((2, RMAX, N), jnp.float32),
            pltpu.SemaphoreType.DMA((2,)),
            pltpu.SemaphoreType.DMA((C,)),
            pltpu.SemaphoreType.DMA((C,)),
            pltpu.SemaphoreType.DMA((C,)),
            pltpu.SemaphoreType.DMA((C,)),
            pltpu.SemaphoreType.DMA((C,)),
        ],
        compiler_params=pltpu.CompilerParams(
            collective_id=0,
            vmem_limit_bytes=40 * 1024 * 1024,
        ),
    )(x)
